# baseline (speedup 1.0000x reference)
import functools

import jax
import jax.numpy as jnp
from jax import lax
from jax.experimental import pallas as pl
from jax.experimental.pallas import tpu as pltpu
from jax.experimental.pallas import tpu_sc as plsc

_F = 26
_V = 100001
_D = 32
_B = 4096

_NC = 2
_NS = 16
_NW = _NC * _NS
_L = 16

_ROWS = _B * _F
_ROWS_W = _ROWS // _NW
_CHUNK = 128
_NCH = _ROWS_W // _CHUNK


def _field_offset_const(chunk: int, group: int):
    base = chunk * _CHUNK + group * _L
    p = base + lax.iota(jnp.int32, _L)
    return (p % _F) * _V


@jax.jit
def _sc_gather(idx2d, table2d):
    mesh = plsc.VectorSubcoreMesh(
        core_axis_name="c", subcore_axis_name="s",
        num_cores=_NC, num_subcores=_NS,
    )

    @functools.partial(
        pl.kernel,
        out_type=jax.ShapeDtypeStruct((_ROWS, _D), jnp.float32),
        mesh=mesh,
        compiler_params=pltpu.CompilerParams(use_tc_tiling_on_sc=False),
        scratch_types=[
            pltpu.VMEM((_ROWS_W,), jnp.int32),
            pltpu.VMEM((_ROWS_W, _D), jnp.float32),
            pltpu.SemaphoreType.DMA,
        ],
    )
    def k(idx_hbm, table_hbm, out_hbm, idx_v, rows_v, sem):
        wid = lax.axis_index("s") * _NC + lax.axis_index("c")
        row0 = wid * _ROWS_W

        pltpu.sync_copy(idx_hbm.at[pl.ds(row0, _ROWS_W)], idx_v)

        for j in range(_NCH):
            for m in range(_CHUNK // _L):
                sl = pl.ds(j * _CHUNK + m * _L, _L)
                idx_v[sl] = idx_v[sl] + _field_offset_const(j, m)

        copies = []
        for j in range(_NCH):
            cp = pltpu.make_async_copy(
                table_hbm.at[idx_v.at[pl.ds(j * _CHUNK, _CHUNK)]],
                rows_v.at[pl.ds(j * _CHUNK, _CHUNK)],
                sem,
            )
            cp.start()
            copies.append(cp)
        for cp in copies:
            cp.wait()

        pltpu.sync_copy(rows_v, out_hbm.at[pl.ds(row0, _ROWS_W)])

    return k(idx2d, table2d)


def kernel(input_tensor, tables):
    idx_flat = input_tensor.reshape(_ROWS)
    table2d = tables.reshape(_F * _V, _D)
    out = _sc_gather(idx_flat, table2d)
    return out.reshape(_B, _F * _D)


# speedup vs baseline: 2.5238x; 2.5238x over previous
import functools

import jax
import jax.numpy as jnp
from jax import lax
from jax.experimental import pallas as pl
from jax.experimental.pallas import tpu as pltpu
from jax.experimental.pallas import tpu_sc as plsc

_F = 26
_V = 100001
_D = 32
_B = 4096

_NC = 2
_NS = 16
_NW = _NC * _NS
_BW = _B // _NW


@jax.jit
def _sc_gather(idx_flat, tables):
    mesh = plsc.VectorSubcoreMesh(
        core_axis_name="c", subcore_axis_name="s",
        num_cores=_NC, num_subcores=_NS,
    )

    @functools.partial(
        pl.kernel,
        out_type=jax.ShapeDtypeStruct((_B, _F * _D), jnp.float32),
        mesh=mesh,
        compiler_params=pltpu.CompilerParams(use_tc_tiling_on_sc=False),
        scratch_types=[
            pltpu.VMEM((_F * _BW,), jnp.int32),
            pltpu.VMEM((_F * _BW, _D), jnp.float32),
            pltpu.SemaphoreType.DMA,
            pltpu.SemaphoreType.DMA,
        ],
    )
    def k(idx_hbm, t_hbm, out_hbm, idx_v, rows_v, sem_i, sem_g):
        wid = lax.axis_index("s") * _NC + lax.axis_index("c")
        b0 = wid * _BW

        idx_cp = []
        for f in range(_F):
            cp = pltpu.make_async_copy(
                idx_hbm.at[pl.ds(f * _B + b0, _BW)],
                idx_v.at[pl.ds(f * _BW, _BW)],
                sem_i,
            )
            cp.start()
            idx_cp.append(cp)
        for cp in idx_cp:
            cp.wait()

        g_cp = []
        for f in range(_F):
            cp = pltpu.make_async_copy(
                t_hbm.at[f].at[idx_v.at[pl.ds(f * _BW, _BW)]],
                rows_v.at[pl.ds(f * _BW, _BW), :],
                sem_g,
            )
            cp.start()
            g_cp.append(cp)
        for cp in g_cp:
            cp.wait()

        for f in range(_F):
            pltpu.sync_copy(
                rows_v.at[pl.ds(f * _BW, _BW), :],
                out_hbm.at[pl.ds(b0, _BW), pl.ds(f * _D, _D)],
            )

    return k(idx_flat, tables)


def kernel(input_tensor, tables):
    idx_flat = input_tensor.T.reshape(_F * _B)
    out = _sc_gather(idx_flat, tables)
    return out
